# baseline (speedup 1.0000x reference)
import jax
import jax.numpy as jnp
from jax import lax
from jax.experimental import pallas as pl
from jax.experimental.pallas import tpu as pltpu
from jax.experimental.pallas import tpu_sc as plsc

N = 1024
E = 16384
DIM = 64
T = 4

NUM_CORES = 2
NUM_SUBCORES = 16
LANES = 16

CBLK = 128
N_CBLK = N // CBLK
ROWS_PER_CORE = N // NUM_CORES
ROWS_PER_TILE = ROWS_PER_CORE // NUM_SUBCORES
EDGES_PER_TILE = E // NUM_SUBCORES
CHUNK = 128
N_CHUNKS = EDGES_PER_TILE // CHUNK
PART_WORDS = ROWS_PER_CORE * N
TILE_WORDS = ROWS_PER_TILE * N
BLK_WORDS = ROWS_PER_CORE * CBLK
TILE_BLK = ROWS_PER_TILE * CBLK
ZBUF = 8192


def _adj_body(edge_hbm, out_hbm, src_v, dst_v, idx_v, val_v, zero_v, part_s):
    c = lax.axis_index("c")
    s = lax.axis_index("s")
    row_lo = c * ROWS_PER_CORE

    zeros16 = jnp.zeros((LANES,), jnp.float32)
    for i in range(ZBUF // LANES):
        zero_v[pl.ds(i * LANES, LANES)] = zeros16
    for i in range(TILE_WORDS // ZBUF):
        pltpu.sync_copy(zero_v,
                        part_s.at[pl.ds(s * TILE_WORDS + i * ZBUF, ZBUF)])
    pltpu.sync_copy(edge_hbm.at[0, pl.ds(s * EDGES_PER_TILE, EDGES_PER_TILE)],
                    src_v)
    pltpu.sync_copy(edge_hbm.at[1, pl.ds(s * EDGES_PER_TILE, EDGES_PER_TILE)],
                    dst_v)

    lane = lax.iota(jnp.int32, LANES)
    for k in range(EDGES_PER_TILE // LANES):
        dummy = s * EDGES_PER_TILE + k * LANES + lane
        src16 = src_v[pl.ds(k * LANES, LANES)]
        dst16 = dst_v[pl.ds(k * LANES, LANES)]
        local = src16 - row_lo
        keep = (local >= 0) & (local < ROWS_PER_CORE)
        g = lax.shift_right_logical(dst16, 7)
        l = lax.bitwise_and(dst16, CBLK - 1)
        flat = g * BLK_WORDS + local * CBLK + l
        flat = jnp.where(keep, flat, dummy)
        val = jnp.where(keep, 1.0, 0.0).astype(jnp.float32)
        j, o = divmod(k, CHUNK // LANES)
        idx_v[j, pl.ds(o * LANES, LANES)] = flat
        val_v[j, pl.ds(o * LANES, LANES)] = val

    plsc.subcore_barrier()

    for j in range(N_CHUNKS):
        pltpu.sync_copy(val_v.at[j], part_s.at[idx_v.at[j]], add=True)

    plsc.subcore_barrier()

    dst_off = s * TILE_WORDS + (s // 2 + c) * BLK_WORDS
    pltpu.sync_copy(part_s.at[pl.ds(s * TILE_WORDS, TILE_WORDS)],
                    out_hbm.at[pl.ds(dst_off, TILE_WORDS)])


_build_adj = pl.kernel(
    _adj_body,
    out_type=jax.ShapeDtypeStruct((N * N,), jnp.float32),
    mesh=plsc.VectorSubcoreMesh(core_axis_name="c", subcore_axis_name="s"),
    scratch_types=[
        pltpu.VMEM((EDGES_PER_TILE,), jnp.int32),
        pltpu.VMEM((EDGES_PER_TILE,), jnp.int32),
        pltpu.VMEM((N_CHUNKS, CHUNK), jnp.int32),
        pltpu.VMEM((N_CHUNKS, CHUNK), jnp.float32),
        pltpu.VMEM((ZBUF,), jnp.float32),
        pltpu.VMEM_SHARED((PART_WORDS,), jnp.float32),
    ],
)


def _dense_body(adjt_hbm, xv_ref, w1_ref, w2_ref, w3_ref, w4_ref, out_ref,
                adj_v, sem):
    f32 = jnp.float32
    dot_t = lambda a, b: lax.dot_general(
        a, b, (((1,), (1,)), ((), ())), preferred_element_type=f32)
    dot_n = lambda a, b: lax.dot_general(
        a, b, (((1,), (0,)), ((), ())), preferred_element_type=f32)
    cp = pltpu.make_async_copy(adjt_hbm, adj_v, sem)
    cp.start()
    w4r = jnp.maximum(w4_ref[...], 0.0)
    c3 = dot_n(w3_ref[...], w4r)
    base = dot_t(xv_ref[...], w1_ref[...])
    cp.wait()
    blocks = [adj_v[pl.ds(g * N, N), :] for g in range(N_CBLK)]
    ones = jnp.ones((CBLK, 1), f32)
    deg = sum(dot_n(b, ones) for b in blocks)
    base = base + dot_t(deg, c3)
    emb = jnp.maximum(base, 0.0)
    for _ in range(T - 1):
        ns = sum(dot_n(blocks[g],
                       lax.slice(emb, (g * CBLK, 0), ((g + 1) * CBLK, DIM)))
                 for g in range(N_CBLK))
        emb = jnp.maximum(base + dot_t(ns, w2_ref[...]), 0.0)
    out_ref[...] = emb


_dense = pl.pallas_call(
    _dense_body,
    out_shape=jax.ShapeDtypeStruct((N, DIM), jnp.float32),
    in_specs=[pl.BlockSpec(memory_space=pltpu.HBM)] + [pl.BlockSpec()] * 5,
    scratch_shapes=[pltpu.VMEM((N_CBLK * N, CBLK), jnp.float32),
                    pltpu.SemaphoreType.DMA],
)


@jax.jit
def kernel(Xv, edge_index, W1, W2, W3, W4):
    adj_t = _build_adj(edge_index).reshape(N_CBLK * N, CBLK)
    return _dense(adj_t, Xv, W1, W2, W3, W4)


# speedup vs baseline: 1.0287x; 1.0287x over previous
import jax
import jax.numpy as jnp
from jax import lax
from jax.experimental import pallas as pl
from jax.experimental.pallas import tpu as pltpu
from jax.experimental.pallas import tpu_sc as plsc

N = 1024
E = 16384
DIM = 64
T = 4

NUM_CORES = 2
NUM_SUBCORES = 16
LANES = 16

CBLK = 128
N_CBLK = N // CBLK
ROWS_PER_CORE = N // NUM_CORES
ROWS_PER_TILE = ROWS_PER_CORE // NUM_SUBCORES
EDGES_PER_TILE = E // NUM_SUBCORES
CHUNK = 128
N_CHUNKS = EDGES_PER_TILE // CHUNK
PART_WORDS = ROWS_PER_CORE * N
TILE_WORDS = ROWS_PER_TILE * N
BLK_WORDS = ROWS_PER_CORE * CBLK
TILE_BLK = ROWS_PER_TILE * CBLK
ZBUF = 8192


def _adj_body(edge_hbm, out_hbm, src_v, dst_v, idx_v, val_v, zero_v, part_s):
    c = lax.axis_index("c")
    s = lax.axis_index("s")
    row_lo = c * ROWS_PER_CORE

    zeros16 = jnp.zeros((LANES,), jnp.float32)
    for i in range(ZBUF // LANES):
        zero_v[pl.ds(i * LANES, LANES)] = zeros16
    for i in range(TILE_WORDS // ZBUF):
        pltpu.sync_copy(zero_v,
                        part_s.at[pl.ds(s * TILE_WORDS + i * ZBUF, ZBUF)])
    pltpu.sync_copy(edge_hbm.at[0, pl.ds(s * EDGES_PER_TILE, EDGES_PER_TILE)],
                    src_v)
    pltpu.sync_copy(edge_hbm.at[1, pl.ds(s * EDGES_PER_TILE, EDGES_PER_TILE)],
                    dst_v)

    lane = lax.iota(jnp.int32, LANES)
    for k in range(EDGES_PER_TILE // LANES):
        dummy = s * EDGES_PER_TILE + k * LANES + lane
        src16 = src_v[pl.ds(k * LANES, LANES)]
        dst16 = dst_v[pl.ds(k * LANES, LANES)]
        local = src16 - row_lo
        keep = (local >= 0) & (local < ROWS_PER_CORE)
        g = lax.shift_right_logical(dst16, 7)
        l = lax.bitwise_and(dst16, CBLK - 1)
        flat = g * BLK_WORDS + local * CBLK + l
        flat = jnp.where(keep, flat, dummy)
        val = jnp.where(keep, 1.0, 0.0).astype(jnp.float32)
        j, o = divmod(k, CHUNK // LANES)
        idx_v[j, pl.ds(o * LANES, LANES)] = flat
        val_v[j, pl.ds(o * LANES, LANES)] = val

    plsc.subcore_barrier()

    for j in range(N_CHUNKS):
        pltpu.sync_copy(val_v.at[j], part_s.at[idx_v.at[j]], add=True)

    plsc.subcore_barrier()

    dst_off = s * TILE_WORDS + (s // 2 + c) * BLK_WORDS
    pltpu.sync_copy(part_s.at[pl.ds(s * TILE_WORDS, TILE_WORDS)],
                    out_hbm.at[pl.ds(dst_off, TILE_WORDS)])


_build_adj = pl.kernel(
    _adj_body,
    out_type=jax.ShapeDtypeStruct((N * N,), jnp.float32),
    mesh=plsc.VectorSubcoreMesh(core_axis_name="c", subcore_axis_name="s"),
    scratch_types=[
        pltpu.VMEM((EDGES_PER_TILE,), jnp.int32),
        pltpu.VMEM((EDGES_PER_TILE,), jnp.int32),
        pltpu.VMEM((N_CHUNKS, CHUNK), jnp.int32),
        pltpu.VMEM((N_CHUNKS, CHUNK), jnp.float32),
        pltpu.VMEM((ZBUF,), jnp.float32),
        pltpu.VMEM_SHARED((PART_WORDS,), jnp.float32),
    ],
)


def _dense_body(adjt_ref, xv_ref, w1_ref, w2_ref, w3_ref, w4_ref, out_ref):
    f32 = jnp.float32
    dot_t = lambda a, b: lax.dot_general(
        a, b, (((1,), (1,)), ((), ())), preferred_element_type=f32)
    dot_n = lambda a, b: lax.dot_general(
        a, b, (((1,), (0,)), ((), ())), preferred_element_type=f32)
    blocks = [adjt_ref[pl.ds(g * N, N), :] for g in range(N_CBLK)]
    w4r = jnp.maximum(w4_ref[...], 0.0)
    c3 = dot_n(w3_ref[...], w4r)
    ones = jnp.ones((CBLK, 1), f32)
    deg = sum(dot_n(b, ones) for b in blocks)
    base = dot_t(xv_ref[...], w1_ref[...]) + dot_t(deg, c3)
    emb = jnp.maximum(base, 0.0)
    for _ in range(T - 1):
        ns = sum(dot_n(blocks[g],
                       lax.slice(emb, (g * CBLK, 0), ((g + 1) * CBLK, DIM)))
                 for g in range(N_CBLK))
        emb = jnp.maximum(base + dot_t(ns, w2_ref[...]), 0.0)
    out_ref[...] = emb


_dense = pl.pallas_call(
    _dense_body,
    out_shape=jax.ShapeDtypeStruct((N, DIM), jnp.float32),
)


@jax.jit
def kernel(Xv, edge_index, W1, W2, W3, W4):
    adj_t = _build_adj(edge_index).reshape(N_CBLK * N, CBLK)
    return _dense(adj_t, Xv, W1, W2, W3, W4)


# speedup vs baseline: 1.0677x; 1.0379x over previous
import jax
import jax.numpy as jnp
from jax import lax
from jax.experimental import pallas as pl
from jax.experimental.pallas import tpu as pltpu
from jax.experimental.pallas import tpu_sc as plsc

N = 1024
E = 16384
DIM = 64
T = 4

NUM_CORES = 2
NUM_SUBCORES = 16
LANES = 16

CBLK = 128
N_CBLK = N // CBLK
ROWS_PER_CORE = N // NUM_CORES
ROWS_PER_TILE = ROWS_PER_CORE // NUM_SUBCORES
EDGES_PER_TILE = E // NUM_SUBCORES
CHUNK = 128
N_CHUNKS = EDGES_PER_TILE // CHUNK
PART_WORDS = ROWS_PER_CORE * N
TILE_WORDS = ROWS_PER_TILE * N
BLK_WORDS = ROWS_PER_CORE * CBLK
TILE_BLK = ROWS_PER_TILE * CBLK
ZBUF = 8192


def _adj_body(edge_hbm, out_hbm, src_v, dst_v, idx_v, val_v, zero_v, sem,
              part_s):
    c = lax.axis_index("c")
    s = lax.axis_index("s")
    row_lo = c * ROWS_PER_CORE

    zeros16 = jnp.zeros((LANES,), jnp.float32)
    for i in range(ZBUF // LANES):
        zero_v[pl.ds(i * LANES, LANES)] = zeros16
    zcps = [pltpu.async_copy(
        zero_v, part_s.at[pl.ds(s * TILE_WORDS + i * ZBUF, ZBUF)], sem)
        for i in range(TILE_WORDS // ZBUF)]
    pltpu.sync_copy(edge_hbm.at[0, pl.ds(s * EDGES_PER_TILE, EDGES_PER_TILE)],
                    src_v)
    pltpu.sync_copy(edge_hbm.at[1, pl.ds(s * EDGES_PER_TILE, EDGES_PER_TILE)],
                    dst_v)

    lane = lax.iota(jnp.int32, LANES)
    for k in range(EDGES_PER_TILE // LANES):
        dummy = s * EDGES_PER_TILE + k * LANES + lane
        src16 = src_v[pl.ds(k * LANES, LANES)]
        dst16 = dst_v[pl.ds(k * LANES, LANES)]
        local = src16 - row_lo
        keep = (local >= 0) & (local < ROWS_PER_CORE)
        g = lax.shift_right_logical(dst16, 7)
        l = lax.bitwise_and(dst16, CBLK - 1)
        flat = g * BLK_WORDS + local * CBLK + l
        flat = jnp.where(keep, flat, dummy)
        val = jnp.where(keep, 1.0, 0.0).astype(jnp.float32)
        j, o = divmod(k, CHUNK // LANES)
        idx_v[j, pl.ds(o * LANES, LANES)] = flat
        val_v[j, pl.ds(o * LANES, LANES)] = val

    for cp in zcps:
        cp.wait()
    plsc.subcore_barrier()

    for j in range(N_CHUNKS):
        pltpu.sync_copy(val_v.at[j], part_s.at[idx_v.at[j]], add=True)

    plsc.subcore_barrier()

    dst_off = s * TILE_WORDS + (s // 2 + c) * BLK_WORDS
    pltpu.sync_copy(part_s.at[pl.ds(s * TILE_WORDS, TILE_WORDS)],
                    out_hbm.at[pl.ds(dst_off, TILE_WORDS)])


_build_adj = pl.kernel(
    _adj_body,
    out_type=jax.ShapeDtypeStruct((N * N,), jnp.float32),
    mesh=plsc.VectorSubcoreMesh(core_axis_name="c", subcore_axis_name="s"),
    scratch_types=[
        pltpu.VMEM((EDGES_PER_TILE,), jnp.int32),
        pltpu.VMEM((EDGES_PER_TILE,), jnp.int32),
        pltpu.VMEM((N_CHUNKS, CHUNK), jnp.int32),
        pltpu.VMEM((N_CHUNKS, CHUNK), jnp.float32),
        pltpu.VMEM((ZBUF,), jnp.float32),
        pltpu.SemaphoreType.DMA,
        pltpu.VMEM_SHARED((PART_WORDS,), jnp.float32),
    ],
)


def _dense_body(adjt_ref, xv_ref, w1_ref, w2_ref, w3_ref, w4_ref, out_ref):
    f32 = jnp.float32
    dot_t = lambda a, b: lax.dot_general(
        a, b, (((1,), (1,)), ((), ())), preferred_element_type=f32)
    dot_n = lambda a, b: lax.dot_general(
        a, b, (((1,), (0,)), ((), ())), preferred_element_type=f32)
    blocks = [adjt_ref[pl.ds(g * N, N), :] for g in range(N_CBLK)]
    w4r = jnp.maximum(w4_ref[...], 0.0)
    c3 = dot_n(w3_ref[...], w4r)
    ones = jnp.ones((CBLK, 1), f32)
    deg = sum(dot_n(b, ones) for b in blocks)
    base = dot_t(xv_ref[...], w1_ref[...]) + dot_t(deg, c3)
    emb = jnp.maximum(base, 0.0)
    for _ in range(T - 1):
        ns = sum(dot_n(blocks[g],
                       lax.slice(emb, (g * CBLK, 0), ((g + 1) * CBLK, DIM)))
                 for g in range(N_CBLK))
        emb = jnp.maximum(base + dot_t(ns, w2_ref[...]), 0.0)
    out_ref[...] = emb


_dense = pl.pallas_call(
    _dense_body,
    out_shape=jax.ShapeDtypeStruct((N, DIM), jnp.float32),
)


@jax.jit
def kernel(Xv, edge_index, W1, W2, W3, W4):
    adj_t = _build_adj(edge_index).reshape(N_CBLK * N, CBLK)
    return _dense(adj_t, Xv, W1, W2, W3, W4)


# speedup vs baseline: 1.0689x; 1.0011x over previous
import jax
import jax.numpy as jnp
from jax import lax
from jax.experimental import pallas as pl
from jax.experimental.pallas import tpu as pltpu
from jax.experimental.pallas import tpu_sc as plsc

N = 1024
E = 16384
DIM = 64
T = 4

NUM_CORES = 2
NUM_SUBCORES = 16
LANES = 16

CBLK = 128
N_CBLK = N // CBLK
ROWS_PER_CORE = N // NUM_CORES
ROWS_PER_TILE = ROWS_PER_CORE // NUM_SUBCORES
EDGES_PER_TILE = E // NUM_SUBCORES
CHUNK = 128
N_CHUNKS = EDGES_PER_TILE // CHUNK
PART_WORDS = ROWS_PER_CORE * N
TILE_WORDS = ROWS_PER_TILE * N
BLK_WORDS = ROWS_PER_CORE * CBLK
TILE_BLK = ROWS_PER_TILE * CBLK
ZBUF = 8192


def _adj_body(edge_hbm, out_hbm, src_v, dst_v, idx_v, val_v, zero_v, sem,
              esem, part_s):
    c = lax.axis_index("c")
    s = lax.axis_index("s")
    row_lo = c * ROWS_PER_CORE

    zeros16 = jnp.zeros((LANES,), jnp.float32)
    for i in range(ZBUF // LANES):
        zero_v[pl.ds(i * LANES, LANES)] = zeros16
    zcps = [pltpu.async_copy(
        zero_v, part_s.at[pl.ds(s * TILE_WORDS + i * ZBUF, ZBUF)], sem)
        for i in range(TILE_WORDS // ZBUF)]
    ecps = [pltpu.async_copy(
        edge_hbm.at[r, pl.ds(s * EDGES_PER_TILE, EDGES_PER_TILE)], buf, esem)
        for r, buf in ((0, src_v), (1, dst_v))]
    for cp in ecps:
        cp.wait()

    lane = lax.iota(jnp.int32, LANES)
    for k in range(EDGES_PER_TILE // LANES):
        dummy = s * EDGES_PER_TILE + k * LANES + lane
        src16 = src_v[pl.ds(k * LANES, LANES)]
        dst16 = dst_v[pl.ds(k * LANES, LANES)]
        local = src16 - row_lo
        keep = (local >= 0) & (local < ROWS_PER_CORE)
        g = lax.shift_right_logical(dst16, 7)
        l = lax.bitwise_and(dst16, CBLK - 1)
        flat = g * BLK_WORDS + local * CBLK + l
        flat = jnp.where(keep, flat, dummy)
        val = jnp.where(keep, 1.0, 0.0).astype(jnp.float32)
        j, o = divmod(k, CHUNK // LANES)
        idx_v[j, pl.ds(o * LANES, LANES)] = flat
        val_v[j, pl.ds(o * LANES, LANES)] = val

    for cp in zcps:
        cp.wait()
    plsc.subcore_barrier()

    for j in range(N_CHUNKS):
        pltpu.sync_copy(val_v.at[j], part_s.at[idx_v.at[j]], add=True)

    plsc.subcore_barrier()

    dst_off = s * TILE_WORDS + (s // 2 + c) * BLK_WORDS
    pltpu.sync_copy(part_s.at[pl.ds(s * TILE_WORDS, TILE_WORDS)],
                    out_hbm.at[pl.ds(dst_off, TILE_WORDS)])


_build_adj = pl.kernel(
    _adj_body,
    out_type=jax.ShapeDtypeStruct((N * N,), jnp.float32),
    mesh=plsc.VectorSubcoreMesh(core_axis_name="c", subcore_axis_name="s"),
    scratch_types=[
        pltpu.VMEM((EDGES_PER_TILE,), jnp.int32),
        pltpu.VMEM((EDGES_PER_TILE,), jnp.int32),
        pltpu.VMEM((N_CHUNKS, CHUNK), jnp.int32),
        pltpu.VMEM((N_CHUNKS, CHUNK), jnp.float32),
        pltpu.VMEM((ZBUF,), jnp.float32),
        pltpu.SemaphoreType.DMA,
        pltpu.SemaphoreType.DMA,
        pltpu.VMEM_SHARED((PART_WORDS,), jnp.float32),
    ],
)


def _dense_body(adjt_ref, xv_ref, w1_ref, w2_ref, w3_ref, w4_ref, out_ref):
    f32 = jnp.float32
    dot_t = lambda a, b: lax.dot_general(
        a, b, (((1,), (1,)), ((), ())), preferred_element_type=f32)
    dot_n = lambda a, b: lax.dot_general(
        a, b, (((1,), (0,)), ((), ())), preferred_element_type=f32)
    blocks = [adjt_ref[pl.ds(g * N, N), :] for g in range(N_CBLK)]
    w4r = jnp.maximum(w4_ref[...], 0.0)
    c3 = dot_n(w3_ref[...], w4r)
    ones = jnp.ones((CBLK, 1), f32)
    deg = sum(dot_n(b, ones) for b in blocks)
    base = dot_t(xv_ref[...], w1_ref[...]) + dot_t(deg, c3)
    emb = jnp.maximum(base, 0.0)
    for _ in range(T - 1):
        ns = sum(dot_n(blocks[g],
                       lax.slice(emb, (g * CBLK, 0), ((g + 1) * CBLK, DIM)))
                 for g in range(N_CBLK))
        emb = jnp.maximum(base + dot_t(ns, w2_ref[...]), 0.0)
    out_ref[...] = emb


_dense = pl.pallas_call(
    _dense_body,
    out_shape=jax.ShapeDtypeStruct((N, DIM), jnp.float32),
)


@jax.jit
def kernel(Xv, edge_index, W1, W2, W3, W4):
    adj_t = _build_adj(edge_index).reshape(N_CBLK * N, CBLK)
    return _dense(adj_t, Xv, W1, W2, W3, W4)
